# Initial kernel scaffold; baseline (speedup 1.0000x reference)
import jax
import jax.numpy as jnp
from jax.experimental import pallas as pl


def kernel(x, adj_t, W1, b1, g1, be1, W2, b2, g2, be2, W3, b3):
    raise NotImplementedError("write your pallas kernel here")


# speedup vs baseline: 10.7250x; 10.7250x over previous
import functools

import jax
import jax.numpy as jnp
from jax import lax
from jax.experimental import pallas as pl
from jax.experimental.pallas import tpu as pltpu
from jax.experimental.pallas import tpu_sc as plsc

_N = 10000
_E = 320000
_D = 128
_DOUT = 40
_NC = 2
_NS = 16
_NW = _NC * _NS
_CHUNK = 128
_NCHUNK = 79
_EPAD = _NW * _NCHUNK * _CHUNK
_R = 10112
_TPR = _R // _NS
_DUMMY = _N


def _sc_agg(d):
    mesh = plsc.VectorSubcoreMesh(core_axis_name="c", subcore_axis_name="s")

    @functools.partial(
        pl.kernel,
        mesh=mesh,
        out_type=jax.ShapeDtypeStruct((_NC, _R, d), jnp.float32),
        scratch_types=[
            pltpu.VMEM((_NCHUNK, _CHUNK), jnp.int32),
            pltpu.VMEM((_NCHUNK, _CHUNK), jnp.int32),
            pltpu.VMEM((_CHUNK, d), jnp.float32),
            pltpu.VMEM_SHARED((_R, d), jnp.float32),
            pltpu.SemaphoreType.DMA,
        ],
    )
    def k(table_hbm, init1_hbm, src_hbm, dst_hbm, out_hbm,
          idx_s, idx_d, rows, acc, sem):
        c = lax.axis_index("c")
        s = lax.axis_index("s")
        wid = s * _NC + c
        pltpu.sync_copy(src_hbm.at[wid], idx_s)
        pltpu.sync_copy(dst_hbm.at[wid], idx_d)
        r0 = s * _TPR

        @pl.when(c == 0)
        def _():
            pltpu.sync_copy(table_hbm.at[pl.ds(r0, _TPR)], acc.at[pl.ds(r0, _TPR)])

        @pl.when(c != 0)
        def _():
            pltpu.sync_copy(init1_hbm.at[pl.ds(r0, _TPR)], acc.at[pl.ds(r0, _TPR)])

        plsc.subcore_barrier()

        def body(j, carry):
            pltpu.async_copy(table_hbm.at[idx_s.at[j]], rows, sem).wait()
            pltpu.sync_copy(rows, acc.at[idx_d.at[j]], add=True)
            return carry

        lax.fori_loop(0, _NCHUNK, body, 0)
        plsc.subcore_barrier()
        pltpu.sync_copy(acc.at[pl.ds(r0, _TPR)], out_hbm.at[c, pl.ds(r0, _TPR)])

    return k


def _sc_deg():
    mesh = plsc.VectorSubcoreMesh(core_axis_name="c", subcore_axis_name="s")
    _EPW = _NCHUNK * _CHUNK
    _NV = _EPW // 16

    @functools.partial(
        pl.kernel,
        mesh=mesh,
        out_type=jax.ShapeDtypeStruct((_NW, _R), jnp.float32),
        scratch_types=[
            pltpu.VMEM((_EPW,), jnp.int32),
            pltpu.VMEM((_R,), jnp.float32),
        ],
        compiler_params=pltpu.CompilerParams(needs_layout_passes=False),
    )
    def k(dst_hbm, out_hbm, idx_d, hist):
        c = lax.axis_index("c")
        s = lax.axis_index("s")
        wid = s * _NC + c
        pltpu.sync_copy(dst_hbm.at[pl.ds(wid * _EPW, _EPW)], idx_d)

        zeros16 = jnp.zeros((16,), jnp.float32)

        def zero_body(i, carry):
            hist[pl.ds(i * 16, 16)] = zeros16
            return carry

        lax.fori_loop(0, _R // 16, zero_body, 0)

        ones16 = jnp.full((16,), 1.0, jnp.float32)

        def body(i, carry):
            v = idx_d[pl.ds(i * 16, 16)]
            plsc.addupdate_scatter(hist, [v], ones16)
            return carry

        lax.fori_loop(0, _NV, body, 0)
        pltpu.sync_copy(hist, out_hbm.at[wid])

    return k


def _dinv_of(degp_ref):
    deg = jnp.sum(degp_ref[...], axis=0) + 1.0
    return lax.rsqrt(deg)


def _tc_prep_body(x_ref, w_ref, degp_ref, out_ref):
    dinv = _dinv_of(degp_ref)
    u = jnp.dot(x_ref[...], w_ref[...], preferred_element_type=jnp.float32)
    out_ref[...] = u * dinv[:, None]


def _tc_bn_body(p_ref, degp_ref, b_ref, g_ref, be_ref, w_ref, out_ref):
    dinv = _dinv_of(degp_ref)
    agg = p_ref[0] + p_ref[1]
    pre = agg * dinv[:, None] + b_ref[...]
    mask = lax.broadcasted_iota(jnp.int32, (_R, 1), 0) < _N
    mu = jnp.sum(jnp.where(mask, pre, 0.0), axis=0) / _N
    var = jnp.sum(jnp.where(mask, (pre - mu) ** 2, 0.0), axis=0) / _N
    y = (pre - mu) * lax.rsqrt(var + 1e-5) * g_ref[...] + be_ref[...]
    r = jnp.maximum(y, 0.0)
    u = jnp.dot(r, w_ref[...], preferred_element_type=jnp.float32)
    out_ref[...] = jnp.where(mask, u * dinv[:, None], 0.0)


def _tc_out_body(p_ref, degp_ref, b_ref, out_ref):
    dinv = _dinv_of(degp_ref)
    agg = p_ref[0] + p_ref[1]
    o = agg[:_N, :_DOUT] * dinv[:_N, None] + b_ref[...]
    m = jnp.max(o, axis=1, keepdims=True)
    o = o - m
    out_ref[...] = o - jnp.log(jnp.sum(jnp.exp(o), axis=1, keepdims=True))


def kernel(x, adj_t, W1, b1, g1, be1, W2, b2, g2, be2, W3, b3):
    src = adj_t[0]
    dst = adj_t[1]
    pad = _EPAD - _E
    fill = jnp.full((pad,), _DUMMY, jnp.int32)
    srcp = jnp.concatenate([src, fill]).reshape(_NW, _NCHUNK, _CHUNK)
    dstp_flat = jnp.concatenate([dst, fill])
    dstp = dstp_flat.reshape(_NW, _NCHUNK, _CHUNK)

    degp = _sc_deg()(dstp_flat)

    xpad = jnp.concatenate([x, jnp.zeros((_R - _N, _D), jnp.float32)])
    h1 = pl.pallas_call(
        _tc_prep_body,
        out_shape=jax.ShapeDtypeStruct((_R, _D), jnp.float32),
    )(xpad, W1, degp)

    z128 = jnp.zeros((_R, _D), jnp.float32)
    p1 = _sc_agg(_D)(h1, z128, srcp, dstp)
    h2 = pl.pallas_call(
        _tc_bn_body,
        out_shape=jax.ShapeDtypeStruct((_R, _D), jnp.float32),
    )(p1, degp, b1, g1, be1, W2)

    p2 = _sc_agg(_D)(h2, z128, srcp, dstp)
    W3p = jnp.concatenate([W3, jnp.zeros((_D, _D - _DOUT), jnp.float32)], axis=1)
    h3 = pl.pallas_call(
        _tc_bn_body,
        out_shape=jax.ShapeDtypeStruct((_R, _D), jnp.float32),
    )(p2, degp, b2, g2, be2, W3p)

    p3 = _sc_agg(_D)(h3, z128, srcp, dstp)
    out = pl.pallas_call(
        _tc_out_body,
        out_shape=jax.ShapeDtypeStruct((_N, _DOUT), jnp.float32),
    )(p3, degp, b3)
    return out
